# baseline (speedup 1.0000x reference)
import functools
import math

import jax
import jax.numpy as jnp
from jax.experimental import pallas as pl
from jax.experimental.pallas import tpu as pltpu

N = 10000
F_IN = 32
DIM = 32
NP0 = 10240
NP1 = 5120
NP2 = 2560
NP3 = 1280
N1 = 5000
N2 = 2500
N3 = 1250

_HIGH = jax.lax.Precision.HIGHEST


def _gcn_body(a_ref, xk_ref, xi_ref, w_ref, dk_ref, di_ref, b_ref, o_ref,
              acc_ref, *, nk, relu, int_a):
    k = pl.program_id(1)

    @pl.when(k == 0)
    def _():
        acc_ref[...] = jnp.zeros_like(acc_ref)

    y = jax.lax.dot_general(xk_ref[...], w_ref[...], (((1,), (0,)), ((), ())),
                            preferred_element_type=jnp.float32,
                            precision=_HIGH) * dk_ref[...]
    if int_a:
        y1 = y.astype(jnp.bfloat16)
        r = y - y1.astype(jnp.float32)
        y2 = r.astype(jnp.bfloat16)
        y3 = (r - y2.astype(jnp.float32)).astype(jnp.bfloat16)
        Y = jnp.concatenate([y1, y2, y3], axis=1)
        a = a_ref[...].astype(jnp.bfloat16)
        p = jax.lax.dot_general(a, Y, (((1,), (0,)), ((), ())),
                                preferred_element_type=jnp.float32)
        acc_ref[...] += p[:, :DIM] + p[:, DIM:2 * DIM] + p[:, 2 * DIM:]
    else:
        a = a_ref[...].astype(jnp.float32)
        acc_ref[...] += jax.lax.dot_general(a, y, (((1,), (0,)), ((), ())),
                                            preferred_element_type=jnp.float32,
                                            precision=_HIGH)

    @pl.when(k == nk - 1)
    def _():
        yi = jax.lax.dot_general(xi_ref[...], w_ref[...],
                                 (((1,), (0,)), ((), ())),
                                 preferred_element_type=jnp.float32,
                                 precision=_HIGH) * di_ref[...]
        o = (acc_ref[...] + yi) * di_ref[...] + b_ref[...]
        if relu:
            o = jnp.maximum(o, 0.0)
        o_ref[...] = o


def _pick_block(d, prefs):
    for p in prefs:
        if d % p == 0:
            return p
    return d


def _gcn(A, x, W, b, dinv, relu, int_a=False):
    n = A.shape[0]
    bm = _pick_block(n, (640,))
    bk = _pick_block(n, (1280,))
    nk = n // bk
    grid = (n // bm, nk)
    dinv2 = dinv.reshape(n, 1)
    b2 = b.reshape(1, DIM)
    body = functools.partial(_gcn_body, nk=nk, relu=relu, int_a=int_a)
    return pl.pallas_call(
        body,
        grid=grid,
        in_specs=[
            pl.BlockSpec((bm, bk), lambda i, k: (i, k)),
            pl.BlockSpec((bk, DIM), lambda i, k: (k, 0)),
            pl.BlockSpec((bm, DIM), lambda i, k: (i, 0)),
            pl.BlockSpec((DIM, DIM), lambda i, k: (0, 0)),
            pl.BlockSpec((bk, 1), lambda i, k: (k, 0)),
            pl.BlockSpec((bm, 1), lambda i, k: (i, 0)),
            pl.BlockSpec((1, DIM), lambda i, k: (0, 0)),
        ],
        out_specs=pl.BlockSpec((bm, DIM), lambda i, k: (i, 0)),
        out_shape=jax.ShapeDtypeStruct((n, DIM), jnp.float32),
        scratch_shapes=[pltpu.VMEM((bm, DIM), jnp.float32)],
    )(A, x, x, W, dinv2, dinv2, b2)


def _mm_body(a_ref, bt_ref, o_ref, acc_ref, *, nk, bm, bn, prec):
    k = pl.program_id(2)
    i = pl.program_id(0)
    j = pl.program_id(1)

    @pl.when(k == 0)
    def _():
        acc_ref[...] = jnp.zeros_like(acc_ref)

    acc_ref[...] += jax.lax.dot_general(
        a_ref[...], bt_ref[...], (((1,), (1,)), ((), ())),
        preferred_element_type=jnp.float32, precision=prec)

    @pl.when(k == nk - 1)
    def _():
        rows = i * bm + jax.lax.broadcasted_iota(jnp.int32, (bm, bn), 0)
        cols = j * bn + jax.lax.broadcasted_iota(jnp.int32, (bm, bn), 1)
        o_ref[...] = jnp.where(rows == cols, 0.0, acc_ref[...])


def _augment_mm(Ap, BTp):
    m, kdim = Ap.shape
    n = BTp.shape[0]
    bm = _pick_block(m, (512, 256))
    bn = bm
    bk = _pick_block(kdim, (2048, 1280))
    nk = kdim // bk
    prec = _HIGH if Ap.dtype == jnp.float32 else jax.lax.Precision.DEFAULT
    body = functools.partial(_mm_body, nk=nk, bm=bm, bn=bn, prec=prec)
    return pl.pallas_call(
        body,
        grid=(m // bm, n // bn, nk),
        in_specs=[
            pl.BlockSpec((bm, bk), lambda i, j, k: (i, k)),
            pl.BlockSpec((bn, bk), lambda i, j, k: (j, k)),
        ],
        out_specs=pl.BlockSpec((bm, bn), lambda i, j, k: (i, j)),
        out_shape=jax.ShapeDtypeStruct((m, n), jnp.float32),
        scratch_shapes=[pltpu.VMEM((bm, bn), jnp.float32)],
    )(Ap, BTp)


def _pool_perm(x, w, n_real, k):
    score = jnp.tanh((x @ w) / jnp.linalg.norm(w))
    idx = jax.lax.broadcasted_iota(jnp.int32, (x.shape[0],), 0)
    score = jnp.where(idx < n_real, score, -jnp.inf)
    vals, perm = jax.lax.top_k(score, k)
    return vals, perm


def _gather_b(M, MT, perm, k, npad, pad_row, out_dtype):
    perm_pad = jnp.concatenate(
        [perm, jnp.full((npad - k,), pad_row, jnp.int32)])
    cols = jax.lax.broadcasted_iota(jnp.int32, (npad, M.shape[1]), 1)
    diag = cols == perm_pad[:, None]
    one = jnp.asarray(1.0, out_dtype)
    Bp = jnp.where(diag, one, M[perm_pad, :].astype(out_dtype))
    BTp = jnp.where(diag, one, MT[perm_pad, :].astype(out_dtype))
    return Bp, BTp


def kernel(x, edge_index, down_w0, down_b0, down_w1, down_b1, down_w2,
           down_b2, down_w3, down_b3, pool_w0, pool_w1, pool_w2, up_w0,
           up_b0, up_w1, up_b1, up_w2, up_b2):
    f32 = jnp.float32
    bf16 = jnp.bfloat16
    src = edge_index[0]
    dst = edge_index[1]

    A0f = jnp.zeros((NP0, NP0), f32).at[dst, src].add(1.0)
    A0Tf = jnp.zeros((NP0, NP0), f32).at[src, dst].add(1.0)
    deg0 = jnp.sum(A0f, axis=1) + 1.0
    dinv0 = jax.lax.rsqrt(deg0)

    xp = jnp.pad(x, ((0, NP0 - N), (0, 0)))
    x1 = _gcn(A0f, xp, down_w0, down_b0, dinv0, relu=True, int_a=True)

    vals1, perm1 = _pool_perm(x1, pool_w0, N, N1)
    x1p = jnp.pad(x1[perm1] * vals1[:, None], ((0, NP1 - N1), (0, 0)))
    Bp1, BTp1 = _gather_b(A0f, A0Tf, perm1, N1, NP1, N, bf16)
    A1 = _augment_mm(Bp1, BTp1)
    deg1 = jnp.sum(A1, axis=1) + 1.0
    dinv1 = jax.lax.rsqrt(deg1)
    x2 = _gcn(A1, x1p, down_w1, down_b1, dinv1, relu=True, int_a=True)

    vals2, perm2 = _pool_perm(x2, pool_w1, N1, N2)
    x2p = jnp.pad(x2[perm2] * vals2[:, None], ((0, NP2 - N2), (0, 0)))
    A1T = A1.T
    Bp2, BTp2 = _gather_b(A1, A1T, perm2, N2, NP2, N1, bf16)
    A2 = _augment_mm(Bp2, BTp2)
    deg2 = jnp.sum(A2, axis=1) + 1.0
    dinv2 = jax.lax.rsqrt(deg2)
    x3 = _gcn(A2, x2p, down_w2, down_b2, dinv2, relu=True)

    vals3, perm3 = _pool_perm(x3, pool_w2, N2, N3)
    x3p = jnp.pad(x3[perm3] * vals3[:, None], ((0, NP3 - N3), (0, 0)))
    A2T = A2.T
    Bp3, BTp3 = _gather_b(A2, A2T, perm3, N3, NP3, N2, f32)
    A3 = _augment_mm(Bp3, BTp3)
    deg3 = jnp.sum(A3, axis=1) + 1.0
    dinv3 = jax.lax.rsqrt(deg3)
    x4 = _gcn(A3, x3p, down_w3, down_b3, dinv3, relu=True)

    up = jnp.zeros((NP2, DIM), f32).at[perm3].set(x4[:N3])
    x5 = _gcn(A2, x3 + up, up_w0, up_b0, dinv2, relu=True)

    up = jnp.zeros((NP1, DIM), f32).at[perm2].set(x5[:N2])
    x6 = _gcn(A1, x2 + up, up_w1, up_b1, dinv1, relu=True, int_a=True)

    up = jnp.zeros((NP0, DIM), f32).at[perm1].set(x6[:N1])
    x7 = _gcn(A0f, x1 + up, up_w2, up_b2, dinv0, relu=False, int_a=True)

    return x7[:N]


# speedup vs baseline: 1.3018x; 1.3018x over previous
import functools
import math

import jax
import jax.numpy as jnp
from jax.experimental import pallas as pl
from jax.experimental.pallas import tpu as pltpu

N = 10000
F_IN = 32
DIM = 32
NP0 = 10240
NP1 = 5120
NP2 = 2560
NP3 = 1280
N1 = 5000
N2 = 2500
N3 = 1250

_HIGH = jax.lax.Precision.HIGHEST


def _gcn_body(a_ref, xk_ref, xi_ref, w_ref, dk_ref, di_ref, b_ref, o_ref,
              acc_ref, *, nk, relu, int_a):
    k = pl.program_id(1)

    @pl.when(k == 0)
    def _():
        acc_ref[...] = jnp.zeros_like(acc_ref)

    y = jax.lax.dot_general(xk_ref[...], w_ref[...], (((1,), (0,)), ((), ())),
                            preferred_element_type=jnp.float32,
                            precision=_HIGH) * dk_ref[...]
    if int_a:
        y1 = y.astype(jnp.bfloat16)
        r = y - y1.astype(jnp.float32)
        y2 = r.astype(jnp.bfloat16)
        y3 = (r - y2.astype(jnp.float32)).astype(jnp.bfloat16)
        Y = jnp.concatenate([y1, y2, y3], axis=1)
        a = a_ref[...].astype(jnp.bfloat16)
        p = jax.lax.dot_general(a, Y, (((1,), (0,)), ((), ())),
                                preferred_element_type=jnp.float32)
        acc_ref[...] += p[:, :DIM] + p[:, DIM:2 * DIM] + p[:, 2 * DIM:]
    else:
        a = a_ref[...].astype(jnp.float32)
        acc_ref[...] += jax.lax.dot_general(a, y, (((1,), (0,)), ((), ())),
                                            preferred_element_type=jnp.float32,
                                            precision=_HIGH)

    @pl.when(k == nk - 1)
    def _():
        yi = jax.lax.dot_general(xi_ref[...], w_ref[...],
                                 (((1,), (0,)), ((), ())),
                                 preferred_element_type=jnp.float32,
                                 precision=_HIGH) * di_ref[...]
        o = (acc_ref[...] + yi) * di_ref[...] + b_ref[...]
        if relu:
            o = jnp.maximum(o, 0.0)
        o_ref[...] = o


def _pick_block(d, prefs):
    for p in prefs:
        if d % p == 0:
            return p
    return d


def _gcn(A, x, W, b, dinv, relu, int_a=False):
    n = A.shape[0]
    bm = _pick_block(n, (640,))
    bk = _pick_block(n, (1280,))
    nk = n // bk
    grid = (n // bm, nk)
    dinv2 = dinv.reshape(n, 1)
    b2 = b.reshape(1, DIM)
    body = functools.partial(_gcn_body, nk=nk, relu=relu, int_a=int_a)
    return pl.pallas_call(
        body,
        grid=grid,
        in_specs=[
            pl.BlockSpec((bm, bk), lambda i, k: (i, k)),
            pl.BlockSpec((bk, DIM), lambda i, k: (k, 0)),
            pl.BlockSpec((bm, DIM), lambda i, k: (i, 0)),
            pl.BlockSpec((DIM, DIM), lambda i, k: (0, 0)),
            pl.BlockSpec((bk, 1), lambda i, k: (k, 0)),
            pl.BlockSpec((bm, 1), lambda i, k: (i, 0)),
            pl.BlockSpec((1, DIM), lambda i, k: (0, 0)),
        ],
        out_specs=pl.BlockSpec((bm, DIM), lambda i, k: (i, 0)),
        out_shape=jax.ShapeDtypeStruct((n, DIM), jnp.float32),
        scratch_shapes=[pltpu.VMEM((bm, DIM), jnp.float32)],
    )(A, x, x, W, dinv2, dinv2, b2)


def _mm_body(a_ref, bt_ref, o_ref, acc_ref, *, nk, bm, bn, prec):
    k = pl.program_id(2)
    i = pl.program_id(0)
    j = pl.program_id(1)

    @pl.when(k == 0)
    def _():
        acc_ref[...] = jnp.zeros_like(acc_ref)

    acc_ref[...] += jax.lax.dot_general(
        a_ref[...], bt_ref[...], (((1,), (1,)), ((), ())),
        preferred_element_type=jnp.float32, precision=prec)

    @pl.when(k == nk - 1)
    def _():
        rows = i * bm + jax.lax.broadcasted_iota(jnp.int32, (bm, bn), 0)
        cols = j * bn + jax.lax.broadcasted_iota(jnp.int32, (bm, bn), 1)
        o_ref[...] = jnp.where(rows == cols, 0.0, acc_ref[...])


def _augment_mm(Ap, BTp):
    m, kdim = Ap.shape
    n = BTp.shape[0]
    bm = _pick_block(m, (512, 256))
    bn = bm
    bk = _pick_block(kdim, (2048, 1280))
    nk = kdim // bk
    prec = _HIGH if Ap.dtype == jnp.float32 else jax.lax.Precision.DEFAULT
    body = functools.partial(_mm_body, nk=nk, bm=bm, bn=bn, prec=prec)
    return pl.pallas_call(
        body,
        grid=(m // bm, n // bn, nk),
        in_specs=[
            pl.BlockSpec((bm, bk), lambda i, j, k: (i, k)),
            pl.BlockSpec((bn, bk), lambda i, j, k: (j, k)),
        ],
        out_specs=pl.BlockSpec((bm, bn), lambda i, j, k: (i, j)),
        out_shape=jax.ShapeDtypeStruct((m, n), jnp.float32),
        scratch_shapes=[pltpu.VMEM((bm, bn), jnp.float32)],
    )(Ap, BTp)


def _pool_perm(x, w, n_real, k):
    score = jnp.tanh((x @ w) / jnp.linalg.norm(w))
    idx = jax.lax.broadcasted_iota(jnp.int32, (x.shape[0],), 0)
    score = jnp.where(idx < n_real, score, -jnp.inf)
    vals, perm = jax.lax.top_k(score, k)
    return vals, perm


def _gather_b(M, MT, perm, k, npad, pad_row, out_dtype):
    perm_pad = jnp.concatenate(
        [perm, jnp.full((npad - k,), pad_row, jnp.int32)])
    cols = jax.lax.broadcasted_iota(jnp.int32, (npad, M.shape[1]), 1)
    diag = cols == perm_pad[:, None]
    one = jnp.asarray(1.0, out_dtype)
    Bp = jnp.where(diag, one, M[perm_pad, :].astype(out_dtype))
    BTp = jnp.where(diag, one, MT[perm_pad, :].astype(out_dtype))
    return Bp, BTp


def kernel(x, edge_index, down_w0, down_b0, down_w1, down_b1, down_w2,
           down_b2, down_w3, down_b3, pool_w0, pool_w1, pool_w2, up_w0,
           up_b0, up_w1, up_b1, up_w2, up_b2):
    f32 = jnp.float32
    bf16 = jnp.bfloat16
    src = edge_index[0]
    dst = edge_index[1]

    A0f = jnp.zeros((NP0, NP0), f32).at[dst, src].add(1.0)
    A0 = A0f.astype(bf16)
    A0T = A0.T
    deg0 = jnp.sum(A0, axis=1, dtype=f32) + 1.0
    dinv0 = jax.lax.rsqrt(deg0)

    xp = jnp.pad(x, ((0, NP0 - N), (0, 0)))
    x1 = _gcn(A0, xp, down_w0, down_b0, dinv0, relu=True, int_a=True)

    vals1, perm1 = _pool_perm(x1, pool_w0, N, N1)
    x1p = jnp.pad(x1[perm1] * vals1[:, None], ((0, NP1 - N1), (0, 0)))
    Bp1, BTp1 = _gather_b(A0, A0T, perm1, N1, NP1, N, bf16)
    A1 = _augment_mm(Bp1, BTp1)
    deg1 = jnp.sum(A1, axis=1) + 1.0
    dinv1 = jax.lax.rsqrt(deg1)
    x2 = _gcn(A1, x1p, down_w1, down_b1, dinv1, relu=True, int_a=True)

    vals2, perm2 = _pool_perm(x2, pool_w1, N1, N2)
    x2p = jnp.pad(x2[perm2] * vals2[:, None], ((0, NP2 - N2), (0, 0)))
    A1T = A1.T
    Bp2, BTp2 = _gather_b(A1, A1T, perm2, N2, NP2, N1, bf16)
    A2 = _augment_mm(Bp2, BTp2)
    deg2 = jnp.sum(A2, axis=1) + 1.0
    dinv2 = jax.lax.rsqrt(deg2)
    x3 = _gcn(A2, x2p, down_w2, down_b2, dinv2, relu=True)

    vals3, perm3 = _pool_perm(x3, pool_w2, N2, N3)
    x3p = jnp.pad(x3[perm3] * vals3[:, None], ((0, NP3 - N3), (0, 0)))
    A2T = A2.T
    Bp3, BTp3 = _gather_b(A2, A2T, perm3, N3, NP3, N2, f32)
    A3 = _augment_mm(Bp3, BTp3)
    deg3 = jnp.sum(A3, axis=1) + 1.0
    dinv3 = jax.lax.rsqrt(deg3)
    x4 = _gcn(A3, x3p, down_w3, down_b3, dinv3, relu=True)

    up = jnp.zeros((NP2, DIM), f32).at[perm3].set(x4[:N3])
    x5 = _gcn(A2, x3 + up, up_w0, up_b0, dinv2, relu=True)

    up = jnp.zeros((NP1, DIM), f32).at[perm2].set(x5[:N2])
    x6 = _gcn(A1, x2 + up, up_w1, up_b1, dinv1, relu=True, int_a=True)

    up = jnp.zeros((NP0, DIM), f32).at[perm1].set(x6[:N1])
    x7 = _gcn(A0, x1 + up, up_w2, up_b2, dinv0, relu=False, int_a=True)

    return x7[:N]


# speedup vs baseline: 1.3986x; 1.0743x over previous
import functools
import math

import jax
import jax.numpy as jnp
from jax.experimental import pallas as pl
from jax.experimental.pallas import tpu as pltpu

N = 10000
F_IN = 32
DIM = 32
NP0 = 10240
NP1 = 5120
NP2 = 2560
NP3 = 1280
N1 = 5000
N2 = 2500
N3 = 1250

_HIGH = jax.lax.Precision.HIGHEST


def _gcn_body(a_ref, xk_ref, xi_ref, w_ref, dk_ref, di_ref, b_ref, o_ref,
              acc_ref, *, nk, relu, int_a):
    k = pl.program_id(1)

    @pl.when(k == 0)
    def _():
        acc_ref[...] = jnp.zeros_like(acc_ref)

    y = jax.lax.dot_general(xk_ref[...], w_ref[...], (((1,), (0,)), ((), ())),
                            preferred_element_type=jnp.float32,
                            precision=_HIGH) * dk_ref[...]
    if int_a:
        y1 = y.astype(jnp.bfloat16)
        r = y - y1.astype(jnp.float32)
        y2 = r.astype(jnp.bfloat16)
        y3 = (r - y2.astype(jnp.float32)).astype(jnp.bfloat16)
        Y = jnp.concatenate([y1, y2, y3], axis=1)
        a = a_ref[...].astype(jnp.bfloat16)
        p = jax.lax.dot_general(a, Y, (((1,), (0,)), ((), ())),
                                preferred_element_type=jnp.float32)
        acc_ref[...] += p[:, :DIM] + p[:, DIM:2 * DIM] + p[:, 2 * DIM:]
    else:
        a = a_ref[...].astype(jnp.float32)
        acc_ref[...] += jax.lax.dot_general(a, y, (((1,), (0,)), ((), ())),
                                            preferred_element_type=jnp.float32,
                                            precision=_HIGH)

    @pl.when(k == nk - 1)
    def _():
        yi = jax.lax.dot_general(xi_ref[...], w_ref[...],
                                 (((1,), (0,)), ((), ())),
                                 preferred_element_type=jnp.float32,
                                 precision=_HIGH) * di_ref[...]
        o = (acc_ref[...] + yi) * di_ref[...] + b_ref[...]
        if relu:
            o = jnp.maximum(o, 0.0)
        o_ref[...] = o


def _pick_block(d, prefs):
    for p in prefs:
        if d % p == 0:
            return p
    return d


def _gcn(A, x, W, b, dinv, relu, int_a=False):
    n = A.shape[0]
    bm = _pick_block(n, (640,))
    bk = _pick_block(n, (1280,))
    nk = n // bk
    grid = (n // bm, nk)
    dinv2 = dinv.reshape(n, 1)
    b2 = b.reshape(1, DIM)
    body = functools.partial(_gcn_body, nk=nk, relu=relu, int_a=int_a)
    return pl.pallas_call(
        body,
        grid=grid,
        in_specs=[
            pl.BlockSpec((bm, bk), lambda i, k: (i, k)),
            pl.BlockSpec((bk, DIM), lambda i, k: (k, 0)),
            pl.BlockSpec((bm, DIM), lambda i, k: (i, 0)),
            pl.BlockSpec((DIM, DIM), lambda i, k: (0, 0)),
            pl.BlockSpec((bk, 1), lambda i, k: (k, 0)),
            pl.BlockSpec((bm, 1), lambda i, k: (i, 0)),
            pl.BlockSpec((1, DIM), lambda i, k: (0, 0)),
        ],
        out_specs=pl.BlockSpec((bm, DIM), lambda i, k: (i, 0)),
        out_shape=jax.ShapeDtypeStruct((n, DIM), jnp.float32),
        scratch_shapes=[pltpu.VMEM((bm, DIM), jnp.float32)],
    )(A, x, x, W, dinv2, dinv2, b2)


def _mm_body(a_ref, bt_ref, o_ref, acc_ref, *, nk, bm, bn, prec):
    k = pl.program_id(2)
    i = pl.program_id(0)
    j = pl.program_id(1)

    @pl.when(k == 0)
    def _():
        acc_ref[...] = jnp.zeros_like(acc_ref)

    acc_ref[...] += jax.lax.dot_general(
        a_ref[...], bt_ref[...], (((1,), (1,)), ((), ())),
        preferred_element_type=jnp.float32, precision=prec)

    @pl.when(k == nk - 1)
    def _():
        rows = i * bm + jax.lax.broadcasted_iota(jnp.int32, (bm, bn), 0)
        cols = j * bn + jax.lax.broadcasted_iota(jnp.int32, (bm, bn), 1)
        o_ref[...] = jnp.where(rows == cols, 0.0, acc_ref[...])


def _augment_mm(Ap, BTp):
    m, kdim = Ap.shape
    n = BTp.shape[0]
    bm = _pick_block(m, (1024, 512, 256))
    bn = bm
    bk = _pick_block(kdim, (2048, 1280))
    nk = kdim // bk
    prec = _HIGH if Ap.dtype == jnp.float32 else jax.lax.Precision.DEFAULT
    body = functools.partial(_mm_body, nk=nk, bm=bm, bn=bn, prec=prec)
    return pl.pallas_call(
        body,
        grid=(m // bm, n // bn, nk),
        in_specs=[
            pl.BlockSpec((bm, bk), lambda i, j, k: (i, k)),
            pl.BlockSpec((bn, bk), lambda i, j, k: (j, k)),
        ],
        out_specs=pl.BlockSpec((bm, bn), lambda i, j, k: (i, j)),
        out_shape=jax.ShapeDtypeStruct((m, n), jnp.float32),
        scratch_shapes=[pltpu.VMEM((bm, bn), jnp.float32)],
    )(Ap, BTp)


def _pool_perm(x, w, n_real, k):
    score = jnp.tanh((x @ w) / jnp.linalg.norm(w))
    idx = jax.lax.broadcasted_iota(jnp.int32, (x.shape[0],), 0)
    score = jnp.where(idx < n_real, score, -jnp.inf)
    vals, perm = jax.lax.top_k(score, k)
    return vals, perm


def _gather_b(M, MT, perm, k, npad, pad_row, out_dtype):
    perm_pad = jnp.concatenate(
        [perm, jnp.full((npad - k,), pad_row, jnp.int32)])
    cols = jax.lax.broadcasted_iota(jnp.int32, (npad, M.shape[1]), 1)
    diag = cols == perm_pad[:, None]
    one = jnp.asarray(1.0, out_dtype)
    Bp = jnp.where(diag, one, M[perm_pad, :].astype(out_dtype))
    BTp = jnp.where(diag, one, MT[perm_pad, :].astype(out_dtype))
    return Bp, BTp


def kernel(x, edge_index, down_w0, down_b0, down_w1, down_b1, down_w2,
           down_b2, down_w3, down_b3, pool_w0, pool_w1, pool_w2, up_w0,
           up_b0, up_w1, up_b1, up_w2, up_b2):
    f32 = jnp.float32
    bf16 = jnp.bfloat16
    src = edge_index[0]
    dst = edge_index[1]

    A0f = jnp.zeros((NP0, NP0), f32).at[dst, src].add(1.0)
    A0 = A0f.astype(bf16)
    A0T = A0.T
    deg0 = jnp.sum(A0, axis=1, dtype=f32) + 1.0
    dinv0 = jax.lax.rsqrt(deg0)

    xp = jnp.pad(x, ((0, NP0 - N), (0, 0)))
    x1 = _gcn(A0, xp, down_w0, down_b0, dinv0, relu=True, int_a=True)

    vals1, perm1 = _pool_perm(x1, pool_w0, N, N1)
    x1p = jnp.pad(x1[perm1] * vals1[:, None], ((0, NP1 - N1), (0, 0)))
    Bp1, BTp1 = _gather_b(A0, A0T, perm1, N1, NP1, N, bf16)
    A1 = _augment_mm(Bp1, BTp1)
    deg1 = jnp.sum(A1, axis=1) + 1.0
    dinv1 = jax.lax.rsqrt(deg1)
    x2 = _gcn(A1, x1p, down_w1, down_b1, dinv1, relu=True, int_a=True)

    vals2, perm2 = _pool_perm(x2, pool_w1, N1, N2)
    x2p = jnp.pad(x2[perm2] * vals2[:, None], ((0, NP2 - N2), (0, 0)))
    A1T = A1.T
    Bp2, BTp2 = _gather_b(A1, A1T, perm2, N2, NP2, N1, bf16)
    A2 = _augment_mm(Bp2, BTp2)
    deg2 = jnp.sum(A2, axis=1) + 1.0
    dinv2 = jax.lax.rsqrt(deg2)
    x3 = _gcn(A2, x2p, down_w2, down_b2, dinv2, relu=True)

    vals3, perm3 = _pool_perm(x3, pool_w2, N2, N3)
    x3p = jnp.pad(x3[perm3] * vals3[:, None], ((0, NP3 - N3), (0, 0)))
    A2T = A2.T
    Bp3, BTp3 = _gather_b(A2, A2T, perm3, N3, NP3, N2, f32)
    A3 = _augment_mm(Bp3, BTp3)
    deg3 = jnp.sum(A3, axis=1) + 1.0
    dinv3 = jax.lax.rsqrt(deg3)
    x4 = _gcn(A3, x3p, down_w3, down_b3, dinv3, relu=True)

    up = jnp.zeros((NP2, DIM), f32).at[perm3].set(x4[:N3])
    x5 = _gcn(A2, x3 + up, up_w0, up_b0, dinv2, relu=True)

    up = jnp.zeros((NP1, DIM), f32).at[perm2].set(x5[:N2])
    x6 = _gcn(A1, x2 + up, up_w1, up_b1, dinv1, relu=True, int_a=True)

    up = jnp.zeros((NP0, DIM), f32).at[perm1].set(x6[:N1])
    x7 = _gcn(A0, x1 + up, up_w2, up_b2, dinv0, relu=False, int_a=True)

    return x7[:N]


# speedup vs baseline: 1.5645x; 1.1186x over previous
import functools
import math

import jax
import jax.numpy as jnp
from jax.experimental import pallas as pl
from jax.experimental.pallas import tpu as pltpu

N = 10000
F_IN = 32
DIM = 32
NP0 = 10240
NP1 = 5120
NP2 = 2560
NP3 = 1280
N1 = 5000
N2 = 2500
N3 = 1250

_HIGH = jax.lax.Precision.HIGHEST


def _gcn_body(a_ref, xk_ref, xi_ref, w_ref, dk_ref, di_ref, b_ref, o_ref,
              acc_ref, *, nk, relu, int_a):
    k = pl.program_id(1)

    @pl.when(k == 0)
    def _():
        acc_ref[...] = jnp.zeros_like(acc_ref)

    y = jax.lax.dot_general(xk_ref[...], w_ref[...], (((1,), (0,)), ((), ())),
                            preferred_element_type=jnp.float32,
                            precision=_HIGH) * dk_ref[...]
    if int_a:
        y1 = y.astype(jnp.bfloat16)
        r = y - y1.astype(jnp.float32)
        y2 = r.astype(jnp.bfloat16)
        y3 = (r - y2.astype(jnp.float32)).astype(jnp.bfloat16)
        Y = jnp.concatenate([y1, y2, y3], axis=1)
        a = a_ref[...].astype(jnp.bfloat16)
        p = jax.lax.dot_general(a, Y, (((1,), (0,)), ((), ())),
                                preferred_element_type=jnp.float32)
        acc_ref[...] += p[:, :DIM] + p[:, DIM:2 * DIM] + p[:, 2 * DIM:]
    else:
        a = a_ref[...].astype(jnp.float32)
        acc_ref[...] += jax.lax.dot_general(a, y, (((1,), (0,)), ((), ())),
                                            preferred_element_type=jnp.float32,
                                            precision=_HIGH)

    @pl.when(k == nk - 1)
    def _():
        yi = jax.lax.dot_general(xi_ref[...], w_ref[...],
                                 (((1,), (0,)), ((), ())),
                                 preferred_element_type=jnp.float32,
                                 precision=_HIGH) * di_ref[...]
        o = (acc_ref[...] + yi) * di_ref[...] + b_ref[...]
        if relu:
            o = jnp.maximum(o, 0.0)
        o_ref[...] = o


def _pick_block(d, prefs):
    for p in prefs:
        if d % p == 0:
            return p
    return d


def _gcn(A, x, W, b, dinv, relu, int_a=False):
    n = A.shape[0]
    bm = _pick_block(n, (1280, 640))
    bk = _pick_block(n, (2560, 1280))
    nk = n // bk
    grid = (n // bm, nk)
    dinv2 = dinv.reshape(n, 1)
    b2 = b.reshape(1, DIM)
    body = functools.partial(_gcn_body, nk=nk, relu=relu, int_a=int_a)
    return pl.pallas_call(
        body,
        grid=grid,
        in_specs=[
            pl.BlockSpec((bm, bk), lambda i, k: (i, k)),
            pl.BlockSpec((bk, DIM), lambda i, k: (k, 0)),
            pl.BlockSpec((bm, DIM), lambda i, k: (i, 0)),
            pl.BlockSpec((DIM, DIM), lambda i, k: (0, 0)),
            pl.BlockSpec((bk, 1), lambda i, k: (k, 0)),
            pl.BlockSpec((bm, 1), lambda i, k: (i, 0)),
            pl.BlockSpec((1, DIM), lambda i, k: (0, 0)),
        ],
        out_specs=pl.BlockSpec((bm, DIM), lambda i, k: (i, 0)),
        out_shape=jax.ShapeDtypeStruct((n, DIM), jnp.float32),
        scratch_shapes=[pltpu.VMEM((bm, DIM), jnp.float32)],
    )(A, x, x, W, dinv2, dinv2, b2)


def _mm_body(a_ref, bt_ref, pa_ref, pb_ref, o_ref, acc_ref, *, nk, bm, bn,
             bk, prec):
    k = pl.program_id(2)
    i = pl.program_id(0)
    j = pl.program_id(1)

    @pl.when(k == 0)
    def _():
        acc_ref[...] = jnp.zeros_like(acc_ref)

    a = a_ref[...]
    one = jnp.ones((), a.dtype)
    ca = k * bk + jax.lax.broadcasted_iota(jnp.int32, (bm, bk), 1)
    a = jnp.where(ca == pa_ref[...], one, a)
    bt = bt_ref[...]
    cb = k * bk + jax.lax.broadcasted_iota(jnp.int32, (bn, bk), 1)
    bt = jnp.where(cb == pb_ref[...], one, bt)

    acc_ref[...] += jax.lax.dot_general(
        a, bt, (((1,), (1,)), ((), ())),
        preferred_element_type=jnp.float32, precision=prec)

    @pl.when(k == nk - 1)
    def _():
        rows = i * bm + jax.lax.broadcasted_iota(jnp.int32, (bm, bn), 0)
        cols = j * bn + jax.lax.broadcasted_iota(jnp.int32, (bm, bn), 1)
        o_ref[...] = jnp.where(rows == cols, 0.0, acc_ref[...])


def _augment_mm(Ap, BTp, perm_pad):
    m, kdim = Ap.shape
    n = BTp.shape[0]
    bm = _pick_block(m, (1024, 512, 256))
    bn = bm
    bk = _pick_block(kdim, (2560, 2048, 1280))
    nk = kdim // bk
    prec = _HIGH if Ap.dtype == jnp.float32 else jax.lax.Precision.DEFAULT
    pp = perm_pad.reshape(m, 1)
    body = functools.partial(_mm_body, nk=nk, bm=bm, bn=bn, bk=bk, prec=prec)
    return pl.pallas_call(
        body,
        grid=(m // bm, n // bn, nk),
        in_specs=[
            pl.BlockSpec((bm, bk), lambda i, j, k: (i, k)),
            pl.BlockSpec((bn, bk), lambda i, j, k: (j, k)),
            pl.BlockSpec((bm, 1), lambda i, j, k: (i, 0)),
            pl.BlockSpec((bn, 1), lambda i, j, k: (j, 0)),
        ],
        out_specs=pl.BlockSpec((bm, bn), lambda i, j, k: (i, j)),
        out_shape=jax.ShapeDtypeStruct((m, n), jnp.float32),
        scratch_shapes=[pltpu.VMEM((bm, bn), jnp.float32)],
    )(Ap, BTp, pp, pp)


def _pool_perm(x, w, n_real, k):
    score = jnp.tanh((x @ w) / jnp.linalg.norm(w))
    idx = jax.lax.broadcasted_iota(jnp.int32, (x.shape[0],), 0)
    score = jnp.where(idx < n_real, score, -jnp.inf)
    vals, perm = jax.lax.top_k(score, k)
    return vals, perm


def _gather_b(M, MT, perm, k, npad, pad_row, out_dtype):
    perm_pad = jnp.concatenate(
        [perm, jnp.full((npad - k,), pad_row, jnp.int32)])
    Bp = M[perm_pad, :].astype(out_dtype)
    BTp = MT[perm_pad, :].astype(out_dtype)
    return Bp, BTp, perm_pad


def kernel(x, edge_index, down_w0, down_b0, down_w1, down_b1, down_w2,
           down_b2, down_w3, down_b3, pool_w0, pool_w1, pool_w2, up_w0,
           up_b0, up_w1, up_b1, up_w2, up_b2):
    f32 = jnp.float32
    bf16 = jnp.bfloat16
    src = edge_index[0]
    dst = edge_index[1]

    A0f = jnp.zeros((NP0, NP0), f32).at[dst, src].add(1.0)
    A0 = A0f.astype(bf16)
    A0T = A0.T
    deg0 = jnp.sum(A0, axis=1, dtype=f32) + 1.0
    dinv0 = jax.lax.rsqrt(deg0)

    xp = jnp.pad(x, ((0, NP0 - N), (0, 0)))
    x1 = _gcn(A0, xp, down_w0, down_b0, dinv0, relu=True, int_a=True)

    vals1, perm1 = _pool_perm(x1, pool_w0, N, N1)
    x1p = jnp.pad(x1[perm1] * vals1[:, None], ((0, NP1 - N1), (0, 0)))
    Bp1, BTp1, pp1 = _gather_b(A0, A0T, perm1, N1, NP1, N, bf16)
    A1 = _augment_mm(Bp1, BTp1, pp1)
    deg1 = jnp.sum(A1, axis=1) + 1.0
    dinv1 = jax.lax.rsqrt(deg1)
    x2 = _gcn(A1, x1p, down_w1, down_b1, dinv1, relu=True, int_a=True)

    vals2, perm2 = _pool_perm(x2, pool_w1, N1, N2)
    x2p = jnp.pad(x2[perm2] * vals2[:, None], ((0, NP2 - N2), (0, 0)))
    A1T = A1.T
    Bp2, BTp2, pp2 = _gather_b(A1, A1T, perm2, N2, NP2, N1, bf16)
    A2 = _augment_mm(Bp2, BTp2, pp2)
    deg2 = jnp.sum(A2, axis=1) + 1.0
    dinv2 = jax.lax.rsqrt(deg2)
    x3 = _gcn(A2, x2p, down_w2, down_b2, dinv2, relu=True)

    vals3, perm3 = _pool_perm(x3, pool_w2, N2, N3)
    x3p = jnp.pad(x3[perm3] * vals3[:, None], ((0, NP3 - N3), (0, 0)))
    A2T = A2.T
    Bp3, BTp3, pp3 = _gather_b(A2, A2T, perm3, N3, NP3, N2, f32)
    A3 = _augment_mm(Bp3, BTp3, pp3)
    deg3 = jnp.sum(A3, axis=1) + 1.0
    dinv3 = jax.lax.rsqrt(deg3)
    x4 = _gcn(A3, x3p, down_w3, down_b3, dinv3, relu=True)

    up = jnp.zeros((NP2, DIM), f32).at[perm3].set(x4[:N3])
    x5 = _gcn(A2, x3 + up, up_w0, up_b0, dinv2, relu=True)

    up = jnp.zeros((NP1, DIM), f32).at[perm2].set(x5[:N2])
    x6 = _gcn(A1, x2 + up, up_w1, up_b1, dinv1, relu=True, int_a=True)

    up = jnp.zeros((NP0, DIM), f32).at[perm1].set(x6[:N1])
    x7 = _gcn(A0, x1 + up, up_w2, up_b2, dinv0, relu=False, int_a=True)

    return x7[:N]


# speedup vs baseline: 1.6064x; 1.0268x over previous
import functools
import math

import jax
import jax.numpy as jnp
from jax.experimental import pallas as pl
from jax.experimental.pallas import tpu as pltpu

N = 10000
F_IN = 32
DIM = 32
NP0 = 10240
NP1 = 5120
NP2 = 2560
NP3 = 1280
N1 = 5000
N2 = 2500
N3 = 1250

_HIGH = jax.lax.Precision.HIGHEST


def _gcn_body(a_ref, xk_ref, xi_ref, w_ref, dk_ref, di_ref, b_ref, o_ref,
              acc_ref, *, nk, relu, int_a):
    k = pl.program_id(1)

    @pl.when(k == 0)
    def _():
        acc_ref[...] = jnp.zeros_like(acc_ref)

    y = jax.lax.dot_general(xk_ref[...], w_ref[...], (((1,), (0,)), ((), ())),
                            preferred_element_type=jnp.float32,
                            precision=_HIGH) * dk_ref[...]
    if int_a:
        y1 = y.astype(jnp.bfloat16)
        r = y - y1.astype(jnp.float32)
        y2 = r.astype(jnp.bfloat16)
        y3 = (r - y2.astype(jnp.float32)).astype(jnp.bfloat16)
        Y = jnp.concatenate([y1, y2, y3], axis=1)
        a = a_ref[...].astype(jnp.bfloat16)
        p = jax.lax.dot_general(a, Y, (((1,), (0,)), ((), ())),
                                preferred_element_type=jnp.float32)
        acc_ref[...] += p[:, :DIM] + p[:, DIM:2 * DIM] + p[:, 2 * DIM:]
    else:
        a = a_ref[...].astype(jnp.float32)
        acc_ref[...] += jax.lax.dot_general(a, y, (((1,), (0,)), ((), ())),
                                            preferred_element_type=jnp.float32,
                                            precision=_HIGH)

    @pl.when(k == nk - 1)
    def _():
        yi = jax.lax.dot_general(xi_ref[...], w_ref[...],
                                 (((1,), (0,)), ((), ())),
                                 preferred_element_type=jnp.float32,
                                 precision=_HIGH) * di_ref[...]
        o = (acc_ref[...] + yi) * di_ref[...] + b_ref[...]
        if relu:
            o = jnp.maximum(o, 0.0)
        o_ref[...] = o


def _pick_block(d, prefs):
    for p in prefs:
        if d % p == 0:
            return p
    return d


def _gcn(A, x, W, b, dinv, relu, int_a=False):
    n = A.shape[0]
    bm = _pick_block(n, (1280, 640))
    bk = _pick_block(n, (2560, 1280))
    nk = n // bk
    grid = (n // bm, nk)
    dinv2 = dinv.reshape(n, 1)
    b2 = b.reshape(1, DIM)
    body = functools.partial(_gcn_body, nk=nk, relu=relu, int_a=int_a)
    return pl.pallas_call(
        body,
        grid=grid,
        in_specs=[
            pl.BlockSpec((bm, bk), lambda i, k: (i, k)),
            pl.BlockSpec((bk, DIM), lambda i, k: (k, 0)),
            pl.BlockSpec((bm, DIM), lambda i, k: (i, 0)),
            pl.BlockSpec((DIM, DIM), lambda i, k: (0, 0)),
            pl.BlockSpec((bk, 1), lambda i, k: (k, 0)),
            pl.BlockSpec((bm, 1), lambda i, k: (i, 0)),
            pl.BlockSpec((1, DIM), lambda i, k: (0, 0)),
        ],
        out_specs=pl.BlockSpec((bm, DIM), lambda i, k: (i, 0)),
        out_shape=jax.ShapeDtypeStruct((n, DIM), jnp.float32),
        scratch_shapes=[pltpu.VMEM((bm, DIM), jnp.float32)],
    )(A, x, x, W, dinv2, dinv2, b2)


def _mm_body(a_ref, bt_ref, pa_ref, pb_ref, o_ref, acc_ref, *, nk, bm, bn,
             bk, prec):
    k = pl.program_id(2)
    i = pl.program_id(0)
    j = pl.program_id(1)

    @pl.when(k == 0)
    def _():
        acc_ref[...] = jnp.zeros_like(acc_ref)

    a = a_ref[...]
    one = jnp.ones((), a.dtype)
    ca = k * bk + jax.lax.broadcasted_iota(jnp.int32, (bm, bk), 1)
    a = jnp.where(ca == pa_ref[...], one, a)
    bt = bt_ref[...]
    cb = k * bk + jax.lax.broadcasted_iota(jnp.int32, (bn, bk), 1)
    bt = jnp.where(cb == pb_ref[...], one, bt)

    acc_ref[...] += jax.lax.dot_general(
        a, bt, (((1,), (1,)), ((), ())),
        preferred_element_type=jnp.float32, precision=prec)

    @pl.when(k == nk - 1)
    def _():
        rows = i * bm + jax.lax.broadcasted_iota(jnp.int32, (bm, bn), 0)
        cols = j * bn + jax.lax.broadcasted_iota(jnp.int32, (bm, bn), 1)
        o_ref[...] = jnp.where(rows == cols, 0.0,
                               acc_ref[...]).astype(o_ref.dtype)


def _augment_mm(Ap, BTp, perm_pad, out_dtype=jnp.float32):
    m, kdim = Ap.shape
    n = BTp.shape[0]
    bm = _pick_block(m, (1024, 512, 256))
    bn = bm
    bk = _pick_block(kdim, (2560, 2048, 1280))
    nk = kdim // bk
    prec = _HIGH if Ap.dtype == jnp.float32 else jax.lax.Precision.DEFAULT
    pp = perm_pad.reshape(m, 1)
    body = functools.partial(_mm_body, nk=nk, bm=bm, bn=bn, bk=bk, prec=prec)
    return pl.pallas_call(
        body,
        grid=(m // bm, n // bn, nk),
        in_specs=[
            pl.BlockSpec((bm, bk), lambda i, j, k: (i, k)),
            pl.BlockSpec((bn, bk), lambda i, j, k: (j, k)),
            pl.BlockSpec((bm, 1), lambda i, j, k: (i, 0)),
            pl.BlockSpec((bn, 1), lambda i, j, k: (j, 0)),
        ],
        out_specs=pl.BlockSpec((bm, bn), lambda i, j, k: (i, j)),
        out_shape=jax.ShapeDtypeStruct((m, n), out_dtype),
        scratch_shapes=[pltpu.VMEM((bm, bn), jnp.float32)],
    )(Ap, BTp, pp, pp)


def _cast_deg_body(a_ref, o_ref, d_ref, dacc_ref, *, nk):
    k = pl.program_id(1)

    @pl.when(k == 0)
    def _():
        dacc_ref[...] = jnp.zeros_like(dacc_ref)

    a = a_ref[...]
    o_ref[...] = a.astype(jnp.bfloat16)
    dacc_ref[...] += jnp.sum(a, axis=1, keepdims=True)

    @pl.when(k == nk - 1)
    def _():
        d_ref[...] = dacc_ref[...] + 1.0


def _cast_deg(Af):
    n = Af.shape[0]
    bm = _pick_block(n, (1280, 640))
    bk = _pick_block(n, (2560, 1280))
    nk = n // bk
    body = functools.partial(_cast_deg_body, nk=nk)
    return pl.pallas_call(
        body,
        grid=(n // bm, nk),
        in_specs=[pl.BlockSpec((bm, bk), lambda i, k: (i, k))],
        out_specs=[
            pl.BlockSpec((bm, bk), lambda i, k: (i, k)),
            pl.BlockSpec((bm, 1), lambda i, k: (i, 0)),
        ],
        out_shape=[
            jax.ShapeDtypeStruct((n, n), jnp.bfloat16),
            jax.ShapeDtypeStruct((n, 1), jnp.float32),
        ],
        scratch_shapes=[pltpu.VMEM((bm, 1), jnp.float32)],
    )(Af)


def _pool_perm(x, w, n_real, k):
    score = jnp.tanh((x @ w) / jnp.linalg.norm(w))
    idx = jax.lax.broadcasted_iota(jnp.int32, (x.shape[0],), 0)
    score = jnp.where(idx < n_real, score, -jnp.inf)
    vals, perm = jax.lax.top_k(score, k)
    return vals, perm


def _gather_b(M, MT, perm, k, npad, pad_row, out_dtype):
    perm_pad = jnp.concatenate(
        [perm, jnp.full((npad - k,), pad_row, jnp.int32)])
    Bp = M[perm_pad, :].astype(out_dtype)
    BTp = MT[perm_pad, :].astype(out_dtype)
    return Bp, BTp, perm_pad


def kernel(x, edge_index, down_w0, down_b0, down_w1, down_b1, down_w2,
           down_b2, down_w3, down_b3, pool_w0, pool_w1, pool_w2, up_w0,
           up_b0, up_w1, up_b1, up_w2, up_b2):
    f32 = jnp.float32
    bf16 = jnp.bfloat16
    src = edge_index[0]
    dst = edge_index[1]

    A0f = jnp.zeros((NP0, NP0), f32).at[dst, src].add(1.0)
    A0, deg0 = _cast_deg(A0f)
    A0T = A0.T
    dinv0 = jax.lax.rsqrt(deg0[:, 0])

    xp = jnp.pad(x, ((0, NP0 - N), (0, 0)))
    x1 = _gcn(A0, xp, down_w0, down_b0, dinv0, relu=True, int_a=True)

    vals1, perm1 = _pool_perm(x1, pool_w0, N, N1)
    x1p = jnp.pad(x1[perm1] * vals1[:, None], ((0, NP1 - N1), (0, 0)))
    Bp1, BTp1, pp1 = _gather_b(A0, A0T, perm1, N1, NP1, N, bf16)
    A1 = _augment_mm(Bp1, BTp1, pp1, out_dtype=bf16)
    deg1 = jnp.sum(A1, axis=1, dtype=f32) + 1.0
    dinv1 = jax.lax.rsqrt(deg1)
    x2 = _gcn(A1, x1p, down_w1, down_b1, dinv1, relu=True, int_a=True)

    vals2, perm2 = _pool_perm(x2, pool_w1, N1, N2)
    x2p = jnp.pad(x2[perm2] * vals2[:, None], ((0, NP2 - N2), (0, 0)))
    A1T = A1.T
    Bp2, BTp2, pp2 = _gather_b(A1, A1T, perm2, N2, NP2, N1, bf16)
    A2 = _augment_mm(Bp2, BTp2, pp2)
    deg2 = jnp.sum(A2, axis=1) + 1.0
    dinv2 = jax.lax.rsqrt(deg2)
    x3 = _gcn(A2, x2p, down_w2, down_b2, dinv2, relu=True)

    vals3, perm3 = _pool_perm(x3, pool_w2, N2, N3)
    x3p = jnp.pad(x3[perm3] * vals3[:, None], ((0, NP3 - N3), (0, 0)))
    A2T = A2.T
    Bp3, BTp3, pp3 = _gather_b(A2, A2T, perm3, N3, NP3, N2, f32)
    A3 = _augment_mm(Bp3, BTp3, pp3)
    deg3 = jnp.sum(A3, axis=1) + 1.0
    dinv3 = jax.lax.rsqrt(deg3)
    x4 = _gcn(A3, x3p, down_w3, down_b3, dinv3, relu=True)

    up = jnp.zeros((NP2, DIM), f32).at[perm3].set(x4[:N3])
    x5 = _gcn(A2, x3 + up, up_w0, up_b0, dinv2, relu=True)

    up = jnp.zeros((NP1, DIM), f32).at[perm2].set(x5[:N2])
    x6 = _gcn(A1, x2 + up, up_w1, up_b1, dinv1, relu=True, int_a=True)

    up = jnp.zeros((NP0, DIM), f32).at[perm1].set(x6[:N1])
    x7 = _gcn(A0, x1 + up, up_w2, up_b2, dinv0, relu=False, int_a=True)

    return x7[:N]
